# baseline (speedup 1.0000x reference)
import functools

import jax
import jax.numpy as jnp
from jax import lax
from jax.experimental import pallas as pl
from jax.experimental.pallas import tpu as pltpu
from jax.experimental.pallas import tpu_sc as plsc

N = 100000
NP = 102400
E = 1600000
EP = 1638400
B = 256
BLK = 1024
GRID = NP // BLK
CH = 128
NROW = EP // CH
SLOTS = 4
PT = NP // 16

_MESH = dict(core_axis_name="c", subcore_axis_name="s")


def _zero_vmem_rows(buf, nrows, width):
    z = jnp.zeros((16,), jnp.float32)
    for i in range(nrows):
        if width == 1:
            buf[pl.ds(i * 16, 16)] = z
        else:
            buf[i] = z


def _make_deg():
    mesh = plsc.VectorSubcoreMesh(**_MESH)
    R = NROW // 32
    RND = R // SLOTS

    @functools.partial(
        pl.kernel, mesh=mesh,
        compiler_params=pltpu.CompilerParams(use_tc_tiling_on_sc=False),
        out_type=jax.ShapeDtypeStruct((2, NP), jnp.float32),
        scratch_types=[
            pltpu.VMEM((2, SLOTS, CH), jnp.int32),
            pltpu.VMEM((CH,), jnp.float32),
            pltpu.VMEM((1600,), jnp.float32),
            pltpu.VMEM_SHARED((NP,), jnp.float32),
            pltpu.SemaphoreType.DMA((2,)),
            pltpu.SemaphoreType.DMA((SLOTS,)),
        ],
    )
    def deg_kernel(dst_hbm, out_hbm, didx_v, ones_v, zbuf_v, hist_sh,
                   isem, ssem):
        c = lax.axis_index("c")
        s = lax.axis_index("s")
        wid = c * 16 + s
        row0 = wid * R
        one = jnp.ones((16,), jnp.float32)
        for i in range(CH // 16):
            ones_v[pl.ds(i * 16, 16)] = one
        _zero_vmem_rows(zbuf_v, 100, 1)
        for k in range(PT // 1600):
            pltpu.sync_copy(zbuf_v, hist_sh.at[pl.ds(s * PT + k * 1600, 1600)])
        plsc.subcore_barrier()

        pltpu.async_copy(dst_hbm.at[pl.ds(row0, SLOTS)], didx_v.at[0],
                         isem.at[0])

        def round_(kk, p, first=False, last=False):
            rbase = row0 + kk * SLOTS
            pltpu.make_async_copy(dst_hbm.at[pl.ds(rbase, SLOTS)],
                                  didx_v.at[p], isem.at[p]).wait()
            for b in range(SLOTS):
                if not first:
                    pltpu.make_async_copy(ones_v,
                                          hist_sh.at[didx_v.at[p, b]],
                                          ssem.at[b]).wait()
                pltpu.async_copy(ones_v, hist_sh.at[didx_v.at[p, b]],
                                 ssem.at[b], add=True)
            if not last:
                nbase = row0 + (kk + 1) * SLOTS
                pltpu.async_copy(dst_hbm.at[pl.ds(nbase, SLOTS)],
                                 didx_v.at[1 - p], isem.at[1 - p])

        def pair(m, carry):
            round_(2 * m + 1, 1)
            round_(2 * m + 2, 0)
            return carry

        round_(0, 0, first=True)
        lax.fori_loop(0, (RND - 2) // 2, pair, 0)
        round_(RND - 1, 1, last=True)
        for b in range(SLOTS):
            pltpu.make_async_copy(ones_v, hist_sh.at[didx_v.at[0, b]],
                                  ssem.at[b]).wait()
        plsc.subcore_barrier()
        pltpu.sync_copy(hist_sh.at[pl.ds(s * PT, PT)],
                        out_hbm.at[c, pl.ds(s * PT, PT)])

    return deg_kernel


def _make_agg(split_edges):
    mesh = plsc.VectorSubcoreMesh(**_MESH)
    R = NROW // 32 if split_edges else NROW // 16
    RND = R // SLOTS

    @functools.partial(
        pl.kernel, mesh=mesh,
        compiler_params=pltpu.CompilerParams(use_tc_tiling_on_sc=False),
        out_type=jax.ShapeDtypeStruct((2, NP, 16), jnp.float32),
        scratch_types=[
            pltpu.VMEM((2, SLOTS, CH), jnp.int32),
            pltpu.VMEM((2, SLOTS, CH), jnp.int32),
            pltpu.VMEM((SLOTS, CH, 16), jnp.float32),
            pltpu.VMEM((128, 16), jnp.float32),
            pltpu.VMEM_SHARED((NP, 16), jnp.float32),
            pltpu.SemaphoreType.DMA((2,)),
            pltpu.SemaphoreType.DMA((2,)),
            pltpu.SemaphoreType.DMA((SLOTS,)),
            pltpu.SemaphoreType.DMA((SLOTS,)),
        ],
    )
    def agg_kernel(src_hbm, dst_hbm, table_hbm, out_hbm,
                   sidx_v, didx_v, rows_v, zbuf_v, acc_sh,
                   ism, idm, gsem, ssem):
        c = lax.axis_index("c")
        s = lax.axis_index("s")
        _zero_vmem_rows(zbuf_v, 128, 16)
        for k in range(PT // 128):
            pltpu.sync_copy(zbuf_v, acc_sh.at[pl.ds(s * PT + k * 128, 128)])
        plsc.subcore_barrier()

        if split_edges:
            row0 = (c * 16 + s) * R
            src_ref = src_hbm
        else:
            row0 = s * R
            src_ref = src_hbm.at[c]

        pltpu.async_copy(src_ref.at[pl.ds(row0, SLOTS)], sidx_v.at[0],
                         ism.at[0])
        pltpu.async_copy(dst_hbm.at[pl.ds(row0, SLOTS)], didx_v.at[0],
                         idm.at[0])

        def round_(kk, p, first=False, last=False):
            rbase = row0 + kk * SLOTS
            pltpu.make_async_copy(src_ref.at[pl.ds(rbase, SLOTS)],
                                  sidx_v.at[p], ism.at[p]).wait()
            pltpu.make_async_copy(dst_hbm.at[pl.ds(rbase, SLOTS)],
                                  didx_v.at[p], idm.at[p]).wait()
            ghandles = []
            for b in range(SLOTS):
                if not first:
                    pltpu.make_async_copy(rows_v.at[b],
                                          acc_sh.at[didx_v.at[p, b]],
                                          ssem.at[b]).wait()
                ghandles.append(pltpu.async_copy(
                    table_hbm.at[sidx_v.at[p, b]], rows_v.at[b], gsem.at[b]))
            if not last:
                nbase = row0 + (kk + 1) * SLOTS
                pltpu.async_copy(src_ref.at[pl.ds(nbase, SLOTS)],
                                 sidx_v.at[1 - p], ism.at[1 - p])
                pltpu.async_copy(dst_hbm.at[pl.ds(nbase, SLOTS)],
                                 didx_v.at[1 - p], idm.at[1 - p])
            for b in range(SLOTS):
                ghandles[b].wait()
                pltpu.async_copy(rows_v.at[b], acc_sh.at[didx_v.at[p, b]],
                                 ssem.at[b], add=True)

        def pair(m, carry):
            round_(2 * m + 1, 1)
            round_(2 * m + 2, 0)
            return carry

        round_(0, 0, first=True)
        lax.fori_loop(0, (RND - 2) // 2, pair, 0)
        round_(RND - 1, 1, last=True)
        for b in range(SLOTS):
            pltpu.make_async_copy(rows_v.at[b], acc_sh.at[didx_v.at[0, b]],
                                  ssem.at[b]).wait()
        plsc.subcore_barrier()
        pltpu.sync_copy(acc_sh.at[pl.ds(s * PT, PT)],
                        out_hbm.at[c, pl.ds(s * PT, PT)])

    return agg_kernel


def _k1_body(h0_ref, h1_ref, x_ref, w1_ref, dinv_ref, g1_ref):
    cnt = h0_ref[0] + h1_ref[0] + 2.0
    dinv = lax.rsqrt(cnt)
    dinv_ref[...] = dinv
    h = (x_ref[:, 0:1] * w1_ref[0:1, :]
         + x_ref[:, 1:2] * w1_ref[1:2, :]
         + x_ref[:, 2:3] * w1_ref[2:3, :])
    g1_ref[...] = h * dinv


def _k2_body(a0_ref, a1_ref, g1_ref, dinv_ref, w2_ref, b1_ref, g2s_ref):
    dinv = dinv_ref[...]
    x2 = dinv * (a0_ref[0] + a1_ref[0] + 2.0 * g1_ref[...]) + b1_ref[...]
    x2 = jnp.maximum(x2, 0.0)
    h2 = jnp.dot(x2, w2_ref[...], preferred_element_type=jnp.float32)
    g2 = h2 * dinv
    g2s_ref[0] = g2[:, :16]
    g2s_ref[1] = g2[:, 16:]


def _k3_body(a2a_ref, a2b_ref, g2a_ref, g2b_ref, dinv_ref, batch_ref,
             b2_ref, w3_ref, b3_ref, out_ref, sums_ref, cnt_ref):
    pid = pl.program_id(0)

    @pl.when(pid == 0)
    def _init():
        sums_ref[...] = jnp.zeros((B, 32), jnp.float32)
        cnt_ref[...] = jnp.zeros((B, 1), jnp.float32)

    dinv = dinv_ref[...]
    x3a = dinv * (a2a_ref[0] + 2.0 * g2a_ref[0]) + b2_ref[:, :16]
    x3b = dinv * (a2b_ref[0] + 2.0 * g2b_ref[0]) + b2_ref[:, 16:]
    x3a = jnp.maximum(x3a, 0.0)
    x3b = jnp.maximum(x3b, 0.0)

    batch = batch_ref[0]
    seg = lax.broadcasted_iota(jnp.int32, (B, BLK), 0)
    row = lax.broadcasted_iota(jnp.int32, (B, BLK), 1) + pid * BLK
    oht = jnp.where((seg == batch) & (row < N), 1.0, 0.0)

    sums_ref[:, :16] += jnp.dot(oht, x3a, preferred_element_type=jnp.float32)
    sums_ref[:, 16:] += jnp.dot(oht, x3b, preferred_element_type=jnp.float32)
    cnt_ref[...] += jnp.sum(oht, axis=1, keepdims=True)

    @pl.when(pid == GRID - 1)
    def _finish():
        mean = sums_ref[...] / jnp.maximum(cnt_ref[...], 1.0)
        out_ref[...] = (
            jnp.dot(mean[:, :16], w3_ref[:16, :],
                    preferred_element_type=jnp.float32)
            + jnp.dot(mean[:, 16:], w3_ref[16:, :],
                      preferred_element_type=jnp.float32)
            + b3_ref[...])


def _col3d_spec(j):
    return pl.BlockSpec((1, BLK, 1), lambda i, j=j: (j * GRID + i, 0, 0))


def _full_spec(shape):
    nd = len(shape)
    return pl.BlockSpec(shape, lambda i: (0,) * nd)


def kernel(x, edge_index, batch, W1, b1, W2, b2, W3, b3):
    f32 = jnp.float32
    pad = EP - E
    src = jnp.concatenate(
        [edge_index[0],
         jnp.arange(pad, dtype=jnp.int32) % N]).reshape(NROW, CH)
    dst = jnp.concatenate(
        [edge_index[1],
         N + (jnp.arange(pad, dtype=jnp.int32) % (NP - N))]).reshape(NROW, CH)
    xp = jnp.pad(x, ((0, NP - N), (0, 0)))
    batch3 = jnp.pad(batch, (0, NP - N)).reshape(GRID, 1, BLK)

    hist = _make_deg()(dst)
    hist3 = hist.reshape(2 * GRID, BLK, 1)

    dinv, g1 = pl.pallas_call(
        _k1_body,
        grid=(GRID,),
        in_specs=[
            _col3d_spec(0), _col3d_spec(1),
            pl.BlockSpec((BLK, 3), lambda i: (i, 0)),
            _full_spec((3, 16)),
        ],
        out_specs=[
            pl.BlockSpec((BLK, 1), lambda i: (i, 0)),
            pl.BlockSpec((BLK, 16), lambda i: (i, 0)),
        ],
        out_shape=[
            jax.ShapeDtypeStruct((NP, 1), f32),
            jax.ShapeDtypeStruct((NP, 16), f32),
        ],
    )(hist3, hist3, xp, W1)

    acc1 = _make_agg(True)(src, dst, g1)

    g2s = pl.pallas_call(
        _k2_body,
        grid=(GRID,),
        in_specs=[
            pl.BlockSpec((1, BLK, 16), lambda i: (0, i, 0)),
            pl.BlockSpec((1, BLK, 16), lambda i: (1, i, 0)),
            pl.BlockSpec((BLK, 16), lambda i: (i, 0)),
            pl.BlockSpec((BLK, 1), lambda i: (i, 0)),
            _full_spec((16, 32)),
            _full_spec((1, 16)),
        ],
        out_specs=pl.BlockSpec((2, BLK, 16), lambda i: (0, i, 0)),
        out_shape=jax.ShapeDtypeStruct((2, NP, 16), f32),
    )(acc1, acc1, g1, dinv, W2, b1.reshape(1, 16))

    table2 = g2s.reshape(2 * NP, 16)
    srcs2 = jnp.stack([src, src + NP])
    acc2 = _make_agg(False)(srcs2, dst, table2)

    out = pl.pallas_call(
        _k3_body,
        grid=(GRID,),
        in_specs=[
            pl.BlockSpec((1, BLK, 16), lambda i: (0, i, 0)),
            pl.BlockSpec((1, BLK, 16), lambda i: (1, i, 0)),
            pl.BlockSpec((1, BLK, 16), lambda i: (0, i, 0)),
            pl.BlockSpec((1, BLK, 16), lambda i: (1, i, 0)),
            pl.BlockSpec((BLK, 1), lambda i: (i, 0)),
            pl.BlockSpec((1, 1, BLK), lambda i: (i, 0, 0)),
            _full_spec((1, 32)),
            _full_spec((32, 32)),
            _full_spec((1, 32)),
        ],
        out_specs=pl.BlockSpec((B, 32), lambda i: (0, 0)),
        out_shape=jax.ShapeDtypeStruct((B, 32), f32),
        scratch_shapes=[
            pltpu.VMEM((B, 32), f32),
            pltpu.VMEM((B, 1), f32),
        ],
    )(acc2, acc2, g2s, g2s, dinv, batch3,
      b2.reshape(1, 32), W3, b3.reshape(1, 32))

    return out


# speedup vs baseline: 1.0961x; 1.0961x over previous
import functools

import jax
import jax.numpy as jnp
from jax import lax
from jax.experimental import pallas as pl
from jax.experimental.pallas import tpu as pltpu
from jax.experimental.pallas import tpu_sc as plsc

N = 100000
NP = 102400
E = 1600000
EP = 1638400
B = 256
BLK = 1024
GRID = NP // BLK
CH = 128
NROW = EP // CH
SLOTS = 10
PT = NP // 16

_MESH = dict(core_axis_name="c", subcore_axis_name="s")


def _zero_vmem_rows(buf, nrows, width):
    z = jnp.zeros((16,), jnp.float32)
    for i in range(nrows):
        if width == 1:
            buf[pl.ds(i * 16, 16)] = z
        else:
            buf[i] = z


def _make_deg():
    mesh = plsc.VectorSubcoreMesh(**_MESH)
    R = NROW // 32
    RND = R // SLOTS

    @functools.partial(
        pl.kernel, mesh=mesh,
        compiler_params=pltpu.CompilerParams(use_tc_tiling_on_sc=False),
        out_type=jax.ShapeDtypeStruct((2, NP), jnp.float32),
        scratch_types=[
            pltpu.VMEM((2, SLOTS, CH), jnp.int32),
            pltpu.VMEM((CH,), jnp.float32),
            pltpu.VMEM((1600,), jnp.float32),
            pltpu.VMEM_SHARED((NP,), jnp.float32),
            pltpu.SemaphoreType.DMA((2,)),
            pltpu.SemaphoreType.DMA((SLOTS,)),
        ],
    )
    def deg_kernel(dst_hbm, out_hbm, didx_v, ones_v, zbuf_v, hist_sh,
                   isem, ssem):
        c = lax.axis_index("c")
        s = lax.axis_index("s")
        wid = c * 16 + s
        row0 = wid * R
        one = jnp.ones((16,), jnp.float32)
        for i in range(CH // 16):
            ones_v[pl.ds(i * 16, 16)] = one
        _zero_vmem_rows(zbuf_v, 100, 1)
        for k in range(PT // 1600):
            pltpu.sync_copy(zbuf_v, hist_sh.at[pl.ds(s * PT + k * 1600, 1600)])
        plsc.subcore_barrier()

        pltpu.async_copy(dst_hbm.at[pl.ds(row0, SLOTS)], didx_v.at[0],
                         isem.at[0])

        def round_(kk, p, first=False, last=False):
            rbase = row0 + kk * SLOTS
            pltpu.make_async_copy(dst_hbm.at[pl.ds(rbase, SLOTS)],
                                  didx_v.at[p], isem.at[p]).wait()
            for b in range(SLOTS):
                if not first:
                    pltpu.make_async_copy(ones_v,
                                          hist_sh.at[didx_v.at[p, b]],
                                          ssem.at[b]).wait()
                pltpu.async_copy(ones_v, hist_sh.at[didx_v.at[p, b]],
                                 ssem.at[b], add=True)
            if not last:
                nbase = row0 + (kk + 1) * SLOTS
                pltpu.async_copy(dst_hbm.at[pl.ds(nbase, SLOTS)],
                                 didx_v.at[1 - p], isem.at[1 - p])

        def pair(m, carry):
            round_(2 * m + 1, 1)
            round_(2 * m + 2, 0)
            return carry

        round_(0, 0, first=True)
        lax.fori_loop(0, (RND - 2) // 2, pair, 0)
        round_(RND - 1, 1, last=True)
        for b in range(SLOTS):
            pltpu.make_async_copy(ones_v, hist_sh.at[didx_v.at[0, b]],
                                  ssem.at[b]).wait()
        plsc.subcore_barrier()
        pltpu.sync_copy(hist_sh.at[pl.ds(s * PT, PT)],
                        out_hbm.at[c, pl.ds(s * PT, PT)])

    return deg_kernel


def _make_agg(split_edges):
    mesh = plsc.VectorSubcoreMesh(**_MESH)
    R = NROW // 32 if split_edges else NROW // 16
    RND = R // SLOTS

    @functools.partial(
        pl.kernel, mesh=mesh,
        compiler_params=pltpu.CompilerParams(use_tc_tiling_on_sc=False),
        out_type=jax.ShapeDtypeStruct((2, NP, 16), jnp.float32),
        scratch_types=[
            pltpu.VMEM((2, SLOTS, CH), jnp.int32),
            pltpu.VMEM((2, SLOTS, CH), jnp.int32),
            pltpu.VMEM((SLOTS, CH, 16), jnp.float32),
            pltpu.VMEM((128, 16), jnp.float32),
            pltpu.VMEM_SHARED((NP, 16), jnp.float32),
            pltpu.SemaphoreType.DMA((2,)),
            pltpu.SemaphoreType.DMA((2,)),
            pltpu.SemaphoreType.DMA((SLOTS,)),
            pltpu.SemaphoreType.DMA((SLOTS,)),
        ],
    )
    def agg_kernel(src_hbm, dst_hbm, table_hbm, out_hbm,
                   sidx_v, didx_v, rows_v, zbuf_v, acc_sh,
                   ism, idm, gsem, ssem):
        c = lax.axis_index("c")
        s = lax.axis_index("s")
        _zero_vmem_rows(zbuf_v, 128, 16)
        for k in range(PT // 128):
            pltpu.sync_copy(zbuf_v, acc_sh.at[pl.ds(s * PT + k * 128, 128)])
        plsc.subcore_barrier()

        if split_edges:
            row0 = (c * 16 + s) * R
            src_ref = src_hbm
        else:
            row0 = s * R
            src_ref = src_hbm.at[c]

        pltpu.async_copy(src_ref.at[pl.ds(row0, SLOTS)], sidx_v.at[0],
                         ism.at[0])
        pltpu.async_copy(dst_hbm.at[pl.ds(row0, SLOTS)], didx_v.at[0],
                         idm.at[0])

        def round_(kk, p, first=False, last=False):
            rbase = row0 + kk * SLOTS
            pltpu.make_async_copy(src_ref.at[pl.ds(rbase, SLOTS)],
                                  sidx_v.at[p], ism.at[p]).wait()
            pltpu.make_async_copy(dst_hbm.at[pl.ds(rbase, SLOTS)],
                                  didx_v.at[p], idm.at[p]).wait()
            ghandles = []
            for b in range(SLOTS):
                if not first:
                    pltpu.make_async_copy(rows_v.at[b],
                                          acc_sh.at[didx_v.at[p, b]],
                                          ssem.at[b]).wait()
                ghandles.append(pltpu.async_copy(
                    table_hbm.at[sidx_v.at[p, b]], rows_v.at[b], gsem.at[b]))
            if not last:
                nbase = row0 + (kk + 1) * SLOTS
                pltpu.async_copy(src_ref.at[pl.ds(nbase, SLOTS)],
                                 sidx_v.at[1 - p], ism.at[1 - p])
                pltpu.async_copy(dst_hbm.at[pl.ds(nbase, SLOTS)],
                                 didx_v.at[1 - p], idm.at[1 - p])
            for b in range(SLOTS):
                ghandles[b].wait()
                pltpu.async_copy(rows_v.at[b], acc_sh.at[didx_v.at[p, b]],
                                 ssem.at[b], add=True)

        def pair(m, carry):
            round_(2 * m + 1, 1)
            round_(2 * m + 2, 0)
            return carry

        round_(0, 0, first=True)
        lax.fori_loop(0, (RND - 2) // 2, pair, 0)
        round_(RND - 1, 1, last=True)
        for b in range(SLOTS):
            pltpu.make_async_copy(rows_v.at[b], acc_sh.at[didx_v.at[0, b]],
                                  ssem.at[b]).wait()
        plsc.subcore_barrier()
        pltpu.sync_copy(acc_sh.at[pl.ds(s * PT, PT)],
                        out_hbm.at[c, pl.ds(s * PT, PT)])

    return agg_kernel


def _k1_body(h0_ref, h1_ref, x_ref, w1_ref, dinv_ref, g1_ref):
    cnt = h0_ref[0] + h1_ref[0] + 2.0
    dinv = lax.rsqrt(cnt)
    dinv_ref[...] = dinv
    h = (x_ref[:, 0:1] * w1_ref[0:1, :]
         + x_ref[:, 1:2] * w1_ref[1:2, :]
         + x_ref[:, 2:3] * w1_ref[2:3, :])
    g1_ref[...] = h * dinv


def _k2_body(a0_ref, a1_ref, g1_ref, dinv_ref, w2_ref, b1_ref, g2s_ref):
    dinv = dinv_ref[...]
    x2 = dinv * (a0_ref[0] + a1_ref[0] + 2.0 * g1_ref[...]) + b1_ref[...]
    x2 = jnp.maximum(x2, 0.0)
    h2 = jnp.dot(x2, w2_ref[...], preferred_element_type=jnp.float32)
    g2 = h2 * dinv
    g2s_ref[0] = g2[:, :16]
    g2s_ref[1] = g2[:, 16:]


def _k3_body(a2a_ref, a2b_ref, g2a_ref, g2b_ref, dinv_ref, batch_ref,
             b2_ref, w3_ref, b3_ref, out_ref, sums_ref, cnt_ref):
    pid = pl.program_id(0)

    @pl.when(pid == 0)
    def _init():
        sums_ref[...] = jnp.zeros((B, 32), jnp.float32)
        cnt_ref[...] = jnp.zeros((B, 1), jnp.float32)

    dinv = dinv_ref[...]
    x3a = dinv * (a2a_ref[0] + 2.0 * g2a_ref[0]) + b2_ref[:, :16]
    x3b = dinv * (a2b_ref[0] + 2.0 * g2b_ref[0]) + b2_ref[:, 16:]
    x3a = jnp.maximum(x3a, 0.0)
    x3b = jnp.maximum(x3b, 0.0)

    batch = batch_ref[0]
    seg = lax.broadcasted_iota(jnp.int32, (B, BLK), 0)
    row = lax.broadcasted_iota(jnp.int32, (B, BLK), 1) + pid * BLK
    oht = jnp.where((seg == batch) & (row < N), 1.0, 0.0)

    sums_ref[:, :16] += jnp.dot(oht, x3a, preferred_element_type=jnp.float32)
    sums_ref[:, 16:] += jnp.dot(oht, x3b, preferred_element_type=jnp.float32)
    cnt_ref[...] += jnp.sum(oht, axis=1, keepdims=True)

    @pl.when(pid == GRID - 1)
    def _finish():
        mean = sums_ref[...] / jnp.maximum(cnt_ref[...], 1.0)
        out_ref[...] = (
            jnp.dot(mean[:, :16], w3_ref[:16, :],
                    preferred_element_type=jnp.float32)
            + jnp.dot(mean[:, 16:], w3_ref[16:, :],
                      preferred_element_type=jnp.float32)
            + b3_ref[...])


def _col3d_spec(j):
    return pl.BlockSpec((1, BLK, 1), lambda i, j=j: (j * GRID + i, 0, 0))


def _full_spec(shape):
    nd = len(shape)
    return pl.BlockSpec(shape, lambda i: (0,) * nd)


def kernel(x, edge_index, batch, W1, b1, W2, b2, W3, b3):
    f32 = jnp.float32
    pad = EP - E
    src = jnp.concatenate(
        [edge_index[0],
         jnp.arange(pad, dtype=jnp.int32) % N]).reshape(NROW, CH)
    dst = jnp.concatenate(
        [edge_index[1],
         N + (jnp.arange(pad, dtype=jnp.int32) % (NP - N))]).reshape(NROW, CH)
    xp = jnp.pad(x, ((0, NP - N), (0, 0)))
    batch3 = jnp.pad(batch, (0, NP - N)).reshape(GRID, 1, BLK)

    hist = _make_deg()(dst)
    hist3 = hist.reshape(2 * GRID, BLK, 1)

    dinv, g1 = pl.pallas_call(
        _k1_body,
        grid=(GRID,),
        in_specs=[
            _col3d_spec(0), _col3d_spec(1),
            pl.BlockSpec((BLK, 3), lambda i: (i, 0)),
            _full_spec((3, 16)),
        ],
        out_specs=[
            pl.BlockSpec((BLK, 1), lambda i: (i, 0)),
            pl.BlockSpec((BLK, 16), lambda i: (i, 0)),
        ],
        out_shape=[
            jax.ShapeDtypeStruct((NP, 1), f32),
            jax.ShapeDtypeStruct((NP, 16), f32),
        ],
    )(hist3, hist3, xp, W1)

    acc1 = _make_agg(True)(src, dst, g1)

    g2s = pl.pallas_call(
        _k2_body,
        grid=(GRID,),
        in_specs=[
            pl.BlockSpec((1, BLK, 16), lambda i: (0, i, 0)),
            pl.BlockSpec((1, BLK, 16), lambda i: (1, i, 0)),
            pl.BlockSpec((BLK, 16), lambda i: (i, 0)),
            pl.BlockSpec((BLK, 1), lambda i: (i, 0)),
            _full_spec((16, 32)),
            _full_spec((1, 16)),
        ],
        out_specs=pl.BlockSpec((2, BLK, 16), lambda i: (0, i, 0)),
        out_shape=jax.ShapeDtypeStruct((2, NP, 16), f32),
    )(acc1, acc1, g1, dinv, W2, b1.reshape(1, 16))

    table2 = g2s.reshape(2 * NP, 16)
    srcs2 = jnp.stack([src, src + NP])
    acc2 = _make_agg(False)(srcs2, dst, table2)

    out = pl.pallas_call(
        _k3_body,
        grid=(GRID,),
        in_specs=[
            pl.BlockSpec((1, BLK, 16), lambda i: (0, i, 0)),
            pl.BlockSpec((1, BLK, 16), lambda i: (1, i, 0)),
            pl.BlockSpec((1, BLK, 16), lambda i: (0, i, 0)),
            pl.BlockSpec((1, BLK, 16), lambda i: (1, i, 0)),
            pl.BlockSpec((BLK, 1), lambda i: (i, 0)),
            pl.BlockSpec((1, 1, BLK), lambda i: (i, 0, 0)),
            _full_spec((1, 32)),
            _full_spec((32, 32)),
            _full_spec((1, 32)),
        ],
        out_specs=pl.BlockSpec((B, 32), lambda i: (0, 0)),
        out_shape=jax.ShapeDtypeStruct((B, 32), f32),
        scratch_shapes=[
            pltpu.VMEM((B, 32), f32),
            pltpu.VMEM((B, 1), f32),
        ],
    )(acc2, acc2, g2s, g2s, dinv, batch3,
      b2.reshape(1, 32), W3, b3.reshape(1, 32))

    return out
